# baseline (speedup 1.0000x reference)
import jax
import jax.numpy as jnp
from jax import lax
from jax.experimental import pallas as pl
from jax.experimental.pallas import tpu as pltpu
from jax.experimental.pallas import tpu_sc as plsc

_B, _T = 4, 4096
_N_TOK = _B * _T
_BIGRAM_DIM = 64
_PACKED_W = 2 * _BIGRAM_DIM
_MODEL_DIM = 1024
_NC, _NS = 2, 16
_NW = _NC * _NS
_TOK_PER_W = _N_TOK // _NW
_LANES = 16
_N_CHUNK = _TOK_PER_W // _LANES
_GCHUNK = 128
_N_GATHER = _TOK_PER_W // _GCHUNK

_MAX_ROWS = 204800
_PACKED_ROWS = _MAX_ROWS // 2
_PACK_BLK = 2048
_PACK_GRID = _MAX_ROWS // _PACK_BLK


def _pack_body(a_ref, out_ref):
    h = pl.program_id(1)

    @pl.when(h == 0)
    def _():
        out_ref[:, 0:_BIGRAM_DIM] = a_ref[...]

    @pl.when(h == 1)
    def _():
        out_ref[:, _BIGRAM_DIM:_PACKED_W] = a_ref[...]


_HALF_BLKS = _PACKED_ROWS // (_PACK_BLK // 2)


@jax.jit
def _pack_table(table):
    return pl.pallas_call(
        _pack_body,
        grid=(_HALF_BLKS, 2),
        in_specs=[
            pl.BlockSpec((_PACK_BLK // 2, _BIGRAM_DIM),
                         lambda i, h: (i + h * _HALF_BLKS, jnp.int32(0))),
        ],
        out_specs=pl.BlockSpec((_PACK_BLK // 2, _PACKED_W),
                               lambda i, h: (i, jnp.int32(0))),
        out_shape=jax.ShapeDtypeStruct((_PACKED_ROWS, _PACKED_W),
                                       jnp.float32),
    )(table)


def _sc_gather_body(ids_hbm, table_hbm, masks_hbm, emb_hbm, mask_hbm,
                    buf_v, tidx_v, hsel_v, rows_v, mrows_v, sem, sem2):
    wid = lax.axis_index("s") * _NC + lax.axis_index("c")
    base = wid * _TOK_PER_W

    buf_v[pl.ds(0, _LANES)] = jnp.zeros((_LANES,), jnp.int32)

    @pl.when(wid % (_T // _TOK_PER_W) != 0)
    def _():
        pltpu.sync_copy(ids_hbm.at[pl.ds(base - 8, 8)], buf_v.at[pl.ds(8, 8)])

    pltpu.sync_copy(ids_hbm.at[pl.ds(base, _TOK_PER_W)],
                    buf_v.at[pl.ds(_LANES, _TOK_PER_W)])

    for j in range(_N_CHUNK):
        v_ids = buf_v[pl.ds(_LANES + _LANES * j, _LANES)]
        v_prev = buf_v[pl.ds(_LANES - 1 + _LANES * j, _LANES)]
        idx = v_prev * 3 + v_ids
        h = 1 + ((idx - _PACKED_ROWS) >> 31)
        tidx_v[pl.ds(_LANES * j, _LANES)] = idx - h * _PACKED_ROWS
        hsel_v[pl.ds(_LANES * j, _LANES)] = h

    copies = []
    for c in range(_N_GATHER):
        copies.append(pltpu.async_copy(
            table_hbm.at[tidx_v.at[pl.ds(c * _GCHUNK, _GCHUNK)]],
            rows_v.at[pl.ds(c * _GCHUNK, _GCHUNK)], sem))
    for c in range(_N_GATHER):
        pltpu.async_copy(
            masks_hbm.at[hsel_v.at[pl.ds(c * _GCHUNK, _GCHUNK)]],
            mrows_v, sem2).wait()
        pltpu.sync_copy(mrows_v,
                        mask_hbm.at[pl.ds(base + c * _GCHUNK, _GCHUNK)])
    for cp in copies:
        cp.wait()

    pltpu.sync_copy(rows_v, emb_hbm.at[pl.ds(base, _TOK_PER_W)])


@jax.jit
def _sc_gather(ids_i32, table_packed, masks):
    mesh = plsc.VectorSubcoreMesh(core_axis_name="c", subcore_axis_name="s")
    return pl.kernel(
        _sc_gather_body,
        out_type=(
            jax.ShapeDtypeStruct((_N_TOK, _PACKED_W), jnp.float32),
            jax.ShapeDtypeStruct((_N_TOK, _PACKED_W), jnp.float32),
        ),
        name="sc_hash_gather",
        mesh=mesh,
        scratch_types=[
            pltpu.VMEM((_LANES + _TOK_PER_W,), jnp.int32),
            pltpu.VMEM((_TOK_PER_W,), jnp.int32),
            pltpu.VMEM((_TOK_PER_W,), jnp.int32),
            pltpu.VMEM((_TOK_PER_W, _PACKED_W), jnp.float32),
            pltpu.VMEM((_GCHUNK, _PACKED_W), jnp.float32),
            pltpu.SemaphoreType.DMA,
            pltpu.SemaphoreType.DMA,
        ],
        compiler_params=pltpu.CompilerParams(use_tc_tiling_on_sc=False),
    )(ids_i32, table_packed, masks)


_TOK_BLOCK = 1024
_BLK_PER_ROW = _T // _TOK_BLOCK


def _proj_body(emb_ref, mask_ref, w_ref, out_ref):
    out_ref[0] = jnp.dot(emb_ref[...] * mask_ref[...], w_ref[...],
                         preferred_element_type=jnp.float32)


@jax.jit
def _project(emb, mask, w_stack):
    return pl.pallas_call(
        _proj_body,
        grid=(_N_TOK // _TOK_BLOCK,),
        in_specs=[
            pl.BlockSpec((_TOK_BLOCK, _PACKED_W),
                         lambda i: (i, jnp.int32(0))),
            pl.BlockSpec((_TOK_BLOCK, _PACKED_W),
                         lambda i: (i, jnp.int32(0))),
            pl.BlockSpec((_PACKED_W, _MODEL_DIM),
                         lambda i: (jnp.int32(0), jnp.int32(0))),
        ],
        out_specs=pl.BlockSpec(
            (1, _TOK_BLOCK, _MODEL_DIM),
            lambda i: (i // _BLK_PER_ROW, i % _BLK_PER_ROW, jnp.int32(0))),
        out_shape=jax.ShapeDtypeStruct((_B, _T, _MODEL_DIM), jnp.float32),
    )(emb, mask, w_stack)


def kernel(ids, table, proj_w):
    ids_i32 = ids.reshape(-1).astype(jnp.int32)
    table_packed = _pack_table(table)
    half = jnp.concatenate(
        [jnp.ones((1, _BIGRAM_DIM), jnp.float32),
         jnp.zeros((1, _BIGRAM_DIM), jnp.float32)], axis=1)
    masks = jnp.concatenate([half, 1.0 - half] + [half] * 6, axis=0)
    w_t = proj_w.T.astype(jnp.float32)
    w_stack = jnp.concatenate([w_t, w_t], axis=0)
    emb2, mask = _sc_gather(ids_i32, table_packed, masks)
    return _project(emb2, mask, w_stack)


# speedup vs baseline: 4.4415x; 4.4415x over previous
import jax
import jax.numpy as jnp
from jax import lax
from jax.experimental import pallas as pl
from jax.experimental.pallas import tpu as pltpu
from jax.experimental.pallas import tpu_sc as plsc

_B, _T = 4, 4096
_N_TOK = _B * _T
_BIGRAM_DIM = 64
_MODEL_DIM = 1024
_NC, _NS = 2, 16
_NW = _NC * _NS
_TOK_PER_W = _N_TOK // _NW
_LANES = 16
_N_CHUNK = _TOK_PER_W // _LANES
_GCHUNK = 128
_N_GATHER = _TOK_PER_W // _GCHUNK


def _sc_gather_body(ids_hbm, table_hbm, emb_hbm, buf_v, idx_v, rows_v, sem):
    wid = lax.axis_index("s") * _NC + lax.axis_index("c")
    base = wid * _TOK_PER_W

    buf_v[pl.ds(0, _LANES)] = jnp.zeros((_LANES,), jnp.int32)

    @pl.when(wid % (_T // _TOK_PER_W) != 0)
    def _():
        pltpu.sync_copy(ids_hbm.at[pl.ds(base - 8, 8)], buf_v.at[pl.ds(8, 8)])

    pltpu.sync_copy(ids_hbm.at[pl.ds(base, _TOK_PER_W)],
                    buf_v.at[pl.ds(_LANES, _TOK_PER_W)])

    for j in range(_N_CHUNK):
        v_ids = buf_v[pl.ds(_LANES + _LANES * j, _LANES)]
        v_prev = buf_v[pl.ds(_LANES - 1 + _LANES * j, _LANES)]
        idx_v[pl.ds(_LANES * j, _LANES)] = 2 * (v_prev * 3 + v_ids)

    copies = []
    for c in range(_N_GATHER):
        copies.append(pltpu.async_copy(
            table_hbm.at[idx_v.at[pl.ds(c * _GCHUNK, _GCHUNK)]],
            rows_v.at[pl.ds(c * _GCHUNK, _GCHUNK)], sem))
    for cp in copies:
        cp.wait()

    pltpu.sync_copy(rows_v, emb_hbm.at[pl.ds(base, _TOK_PER_W)])


@jax.jit
def _sc_gather(ids_i32, table):
    mesh = plsc.VectorSubcoreMesh(core_axis_name="c", subcore_axis_name="s")
    return pl.kernel(
        _sc_gather_body,
        out_type=jax.ShapeDtypeStruct((_N_TOK, _BIGRAM_DIM), jnp.float32),
        name="sc_hash_gather",
        mesh=mesh,
        scratch_types=[
            pltpu.VMEM((_LANES + _TOK_PER_W,), jnp.int32),
            pltpu.VMEM((_TOK_PER_W,), jnp.int32),
            pltpu.VMEM((_TOK_PER_W, _BIGRAM_DIM), jnp.float32),
            pltpu.SemaphoreType.DMA,
        ],
        compiler_params=pltpu.CompilerParams(use_tc_tiling_on_sc=False),
    )(ids_i32, table)


_TOK_BLOCK = 1024


def _proj_body(emb_ref, w_ref, out_ref):
    out_ref[0] = jnp.dot(emb_ref[...], w_ref[...],
                         preferred_element_type=jnp.float32)


_BLK_PER_ROW = _T // _TOK_BLOCK


@jax.jit
def _project(emb, proj_w_t):
    return pl.pallas_call(
        _proj_body,
        grid=(_N_TOK // _TOK_BLOCK,),
        in_specs=[
            pl.BlockSpec((_TOK_BLOCK, _BIGRAM_DIM),
                         lambda i: (i, jnp.int32(0))),
            pl.BlockSpec((_BIGRAM_DIM, _MODEL_DIM),
                         lambda i: (jnp.int32(0), jnp.int32(0))),
        ],
        out_specs=pl.BlockSpec(
            (1, _TOK_BLOCK, _MODEL_DIM),
            lambda i: (i // _BLK_PER_ROW, i % _BLK_PER_ROW, jnp.int32(0))),
        out_shape=jax.ShapeDtypeStruct((_B, _T, _MODEL_DIM), jnp.float32),
    )(emb, proj_w_t)


_MAX_ROWS = 201056


def kernel(ids, table, proj_w):
    ids_i32 = ids.reshape(-1).astype(jnp.int32)
    tp = jnp.concatenate(
        [table[:_MAX_ROWS], jnp.zeros((_MAX_ROWS, _BIGRAM_DIM),
                                      jnp.float32)], axis=1)
    table_lin = tp.reshape(2 * _MAX_ROWS, _BIGRAM_DIM)
    emb = _sc_gather(ids_i32, table_lin)
    return _project(emb, proj_w.T)


# speedup vs baseline: 4.5413x; 1.0225x over previous
import jax
import jax.numpy as jnp
from jax import lax
from jax.experimental import pallas as pl
from jax.experimental.pallas import tpu as pltpu
from jax.experimental.pallas import tpu_sc as plsc

_B, _T = 4, 4096
_N_TOK = _B * _T
_BIGRAM_DIM = 64
_MODEL_DIM = 1024
_NC, _NS = 2, 16
_NW = _NC * _NS
_TOK_PER_W = _N_TOK // _NW
_LANES = 16
_N_CHUNK = _TOK_PER_W // _LANES
_GCHUNK = 128
_N_GATHER = _TOK_PER_W // _GCHUNK


def _sc_gather_body(ids_hbm, table_hbm, emb_hbm, buf_v, idx_v, rows_v, sem):
    wid = lax.axis_index("s") * _NC + lax.axis_index("c")
    base = wid * _TOK_PER_W

    buf_v[pl.ds(0, _LANES)] = jnp.zeros((_LANES,), jnp.int32)

    @pl.when(wid % (_T // _TOK_PER_W) != 0)
    def _():
        pltpu.sync_copy(ids_hbm.at[pl.ds(base - 8, 8)], buf_v.at[pl.ds(8, 8)])

    pltpu.sync_copy(ids_hbm.at[pl.ds(base, _TOK_PER_W)],
                    buf_v.at[pl.ds(_LANES, _TOK_PER_W)])

    for j in range(_N_CHUNK):
        v_ids = buf_v[pl.ds(_LANES + _LANES * j, _LANES)]
        v_prev = buf_v[pl.ds(_LANES - 1 + _LANES * j, _LANES)]
        idx_v[pl.ds(_LANES * j, _LANES)] = 2 * (v_prev * 3 + v_ids)

    copies = []
    for c in range(_N_GATHER):
        copies.append(pltpu.async_copy(
            table_hbm.at[idx_v.at[pl.ds(c * _GCHUNK, _GCHUNK)]],
            rows_v.at[pl.ds(c * _GCHUNK, _GCHUNK)], sem))
    for cp in copies:
        cp.wait()

    pltpu.sync_copy(rows_v, emb_hbm.at[pl.ds(base, _TOK_PER_W)])


@jax.jit
def _sc_gather(ids_i32, table):
    mesh = plsc.VectorSubcoreMesh(core_axis_name="c", subcore_axis_name="s")
    return pl.kernel(
        _sc_gather_body,
        out_type=jax.ShapeDtypeStruct((_N_TOK, _BIGRAM_DIM), jnp.float32),
        name="sc_hash_gather",
        mesh=mesh,
        scratch_types=[
            pltpu.VMEM((_LANES + _TOK_PER_W,), jnp.int32),
            pltpu.VMEM((_TOK_PER_W,), jnp.int32),
            pltpu.VMEM((_TOK_PER_W, _BIGRAM_DIM), jnp.float32),
            pltpu.SemaphoreType.DMA,
        ],
        compiler_params=pltpu.CompilerParams(use_tc_tiling_on_sc=False),
    )(ids_i32, table)


_TOK_BLOCK = 2048


def _proj_body(emb_ref, w_ref, out_ref):
    out_ref[0] = jnp.dot(emb_ref[...], w_ref[...],
                         preferred_element_type=jnp.float32)


_BLK_PER_ROW = _T // _TOK_BLOCK


@jax.jit
def _project(emb, proj_w_t):
    return pl.pallas_call(
        _proj_body,
        grid=(_N_TOK // _TOK_BLOCK,),
        in_specs=[
            pl.BlockSpec((_TOK_BLOCK, _BIGRAM_DIM),
                         lambda i: (i, jnp.int32(0))),
            pl.BlockSpec((_BIGRAM_DIM, _MODEL_DIM),
                         lambda i: (jnp.int32(0), jnp.int32(0))),
        ],
        out_specs=pl.BlockSpec(
            (1, _TOK_BLOCK, _MODEL_DIM),
            lambda i: (i // _BLK_PER_ROW, i % _BLK_PER_ROW, jnp.int32(0))),
        out_shape=jax.ShapeDtypeStruct((_B, _T, _MODEL_DIM), jnp.float32),
    )(emb, proj_w_t)


_MAX_ROWS = 201056


def kernel(ids, table, proj_w):
    ids_i32 = ids.reshape(-1).astype(jnp.int32)
    tp = jnp.pad(table[:_MAX_ROWS], ((0, 0), (0, _BIGRAM_DIM)))
    table_lin = tp.reshape(2 * _MAX_ROWS, _BIGRAM_DIM)
    emb = _sc_gather(ids_i32, table_lin)
    return _project(emb, proj_w.T)
